# Initial kernel scaffold; baseline (speedup 1.0000x reference)
import jax
import jax.numpy as jnp
from jax.experimental import pallas as pl


def kernel(x, channel_choice, bn1_beta, bn1_gamma, bn1_mean, bn1_var, bn2_beta, bn2_gamma, bn2_mean, bn2_var, bn3_beta, bn3_gamma, bn3_mean, bn3_var, w1, w3, wd):
    raise NotImplementedError("write your pallas kernel here")


# speedup vs baseline: 8.8684x; 8.8684x over previous
import functools

import jax
import jax.numpy as jnp
from jax import lax
from jax.experimental import pallas as pl
from jax.experimental.pallas import tpu as pltpu

_EPS = 1e-5
_VMEM_LIMIT = 64 * 1024 * 1024


def _fused_block_kernel(x_ref, bw_ref, bb_ref, wd_ref, b2_ref, w3_ref, b3_ref,
                        o_ref, *, half, mid, H, W, pad):
    L = H * W
    xb = x_ref[0]
    y = jnp.dot(bw_ref[...], xb, preferred_element_type=jnp.float32)
    y = y + bb_ref[...]
    o_ref[0, :half, :] = y[:half]
    h1 = jnp.maximum(y[half:], 0.0)

    zp = jnp.zeros((mid, pad), jnp.float32)
    hp = jnp.concatenate([zp, h1, zp], axis=1)
    wcol = lax.broadcasted_iota(jnp.int32, (1, L), 1) % W
    mask_l = (wcol != 0).astype(jnp.float32)
    mask_r = (wcol != W - 1).astype(jnp.float32)
    acc = jnp.zeros((mid, L), jnp.float32)
    for dh in (-1, 0, 1):
        for dw in (-1, 0, 1):
            t = 3 * (dh + 1) + (dw + 1)
            s = pad + dh * W + dw
            tap = hp[:, s:s + L]
            if dw == -1:
                tap = tap * mask_l
            elif dw == 1:
                tap = tap * mask_r
            acc = acc + tap * wd_ref[:, t:t + 1]
    h2 = acc + b2_ref[...]

    out = jnp.dot(w3_ref[...], h2, preferred_element_type=jnp.float32)
    o_ref[0, half:, :] = jnp.maximum(out + b3_ref[...], 0.0)


def _bn_fold(gamma, beta, mean, var):
    s = gamma * lax.rsqrt(var + _EPS)
    return s, beta - mean * s


def kernel(x, channel_choice, bn1_beta, bn1_gamma, bn1_mean, bn1_var,
           bn2_beta, bn2_gamma, bn2_mean, bn2_var,
           bn3_beta, bn3_gamma, bn3_mean, bn3_var,
           w1, w3, wd):
    B, C, H, W = x.shape
    mid = w1.shape[0]
    outputs = w3.shape[0]
    L = H * W

    s1, b1 = _bn_fold(bn1_gamma, bn1_beta, bn1_mean, bn1_var)
    s2, b2 = _bn_fold(bn2_gamma, bn2_beta, bn2_mean, bn2_var)
    s3, b3 = _bn_fold(bn3_gamma, bn3_beta, bn3_mean, bn3_var)

    mask = channel_choice[0, :mid]
    w1_eff = w1 * (mask * s1)[:, None]

    half = C // 2
    sel = jnp.zeros((half, C), jnp.float32).at[
        jnp.arange(half), 2 * jnp.arange(half)].set(1.0)
    w1_big = jnp.zeros((mid, C), jnp.float32).at[:, 1::2].set(w1_eff)
    big_w = jnp.concatenate([sel, w1_big], axis=0)
    big_b = jnp.concatenate([jnp.zeros((half,), jnp.float32), b1])[:, None]

    wd_t = (wd * s2[None, :]).T
    w3_eff = w3 * s3[:, None]

    x3 = x.reshape(B, C, L)
    pad = 32
    kern = functools.partial(_fused_block_kernel, half=half, mid=mid, H=H,
                             W=W, pad=pad)
    out = pl.pallas_call(
        kern,
        out_shape=jax.ShapeDtypeStruct((B, half + outputs, L), jnp.float32),
        grid_spec=pltpu.PrefetchScalarGridSpec(
            num_scalar_prefetch=0,
            grid=(B,),
            in_specs=[
                pl.BlockSpec((1, C, L), lambda b: (b, 0, 0)),
                pl.BlockSpec((half + mid, C), lambda b: (0, 0)),
                pl.BlockSpec((half + mid, 1), lambda b: (0, 0)),
                pl.BlockSpec((mid, 9), lambda b: (0, 0)),
                pl.BlockSpec((mid, 1), lambda b: (0, 0)),
                pl.BlockSpec((outputs, mid), lambda b: (0, 0)),
                pl.BlockSpec((outputs, 1), lambda b: (0, 0)),
            ],
            out_specs=pl.BlockSpec((1, half + outputs, L), lambda b: (b, 0, 0)),
        ),
        compiler_params=pltpu.CompilerParams(
            dimension_semantics=("parallel",),
            vmem_limit_bytes=_VMEM_LIMIT,
        ),
    )(x3, big_w, big_b, wd_t, b2[:, None], w3_eff, b3[:, None])
    return out.reshape(B, half + outputs, H, W)
